# Initial kernel scaffold; baseline (speedup 1.0000x reference)
import jax
import jax.numpy as jnp
from jax.experimental import pallas as pl


def kernel(z_e, W):
    raise NotImplementedError("write your pallas kernel here")


# speedup vs baseline: 1.8380x; 1.8380x over previous
import jax
import jax.numpy as jnp
from jax import lax
from jax.experimental import pallas as pl
from jax.experimental.pallas import tpu as pltpu

N_TOK = 16384
N_CODE = 1024
DIM = 256
BZ = 512
NB = N_TOK // BZ


def _vq_body(z_ref, w_ref, zq_ref, idx_ref, loss_ref):
    z = z_ref[...]
    w = w_ref[...]
    zsq = jnp.sum(z * z, axis=1, keepdims=True)
    wsq = jnp.sum(w * w, axis=1)
    cross = lax.dot_general(z, w, (((1,), (1,)), ((), ())),
                            preferred_element_type=jnp.float32)
    dist = zsq + wsq[None, :] - 2.0 * cross
    minval = jnp.min(dist, axis=1, keepdims=True)
    col = lax.broadcasted_iota(jnp.int32, (BZ, N_CODE), 1)
    idx = jnp.min(jnp.where(dist == minval, col, N_CODE), axis=1)
    idx_ref[0, 0, :] = idx
    onehot = (col == idx[:, None]).astype(jnp.float32)
    zq_ref[...] = lax.dot_general(onehot, w, (((1,), (0,)), ((), ())),
                                  preferred_element_type=jnp.float32)

    part = jnp.sum(minval).reshape(1, 1)

    @pl.when(pl.program_id(0) == 0)
    def _():
        loss_ref[...] = jnp.zeros((1, 1), jnp.float32)

    loss_ref[...] += part


def kernel(z_e, W):
    z_q, idx3, loss = pl.pallas_call(
        _vq_body,
        grid=(NB,),
        in_specs=[
            pl.BlockSpec((BZ, DIM), lambda i: (i, 0)),
            pl.BlockSpec((N_CODE, DIM), lambda i: (0, 0)),
        ],
        out_specs=[
            pl.BlockSpec((BZ, DIM), lambda i: (i, 0)),
            pl.BlockSpec((1, 1, BZ), lambda i: (i, 0, 0)),
            pl.BlockSpec((1, 1), lambda i: (0, 0)),
        ],
        out_shape=[
            jax.ShapeDtypeStruct((N_TOK, DIM), jnp.float32),
            jax.ShapeDtypeStruct((NB, 1, BZ), jnp.int32),
            jax.ShapeDtypeStruct((1, 1), jnp.float32),
        ],
    )(z_e, W)
    indices = idx3.reshape(N_TOK)
    vq_loss = loss[0, 0] * (1.25 / (N_TOK * DIM))
    return (z_q, indices, vq_loss)
